# baseline (speedup 1.0000x reference)
import functools

import jax
import jax.numpy as jnp
from jax import lax
from jax.experimental import pallas as pl
from jax.experimental.pallas import tpu as pltpu
from jax.experimental.pallas import tpu_sc as plsc

B = 1024
NF = 26
V = 1000
D = 16
S = NF * V

NCORE = 2
NSUB = 16
D_PER_CORE = D // NCORE
PAIRS_PER_CORE = NF * D_PER_CORE
PAIRS_PER_W = PAIRS_PER_CORE // NSUB
CHUNKS = B // 16
COLS_PER_W = B // NSUB

_sc_mesh = plsc.VectorSubcoreMesh(core_axis_name="c", subcore_axis_name="s")


def _sc_body(xt_hbm, tbl_hbm, emb_hbm, fmp_hbm,
             trows_v, xrows_v, embuf_v, vs_v, vq_v,
             accs_v, accq_v, tmp_v, shs_v, shq_v, sem):
    cid = lax.axis_index("c")
    sid = lax.axis_index("s")
    zero16 = jnp.zeros((16,), jnp.float32)

    copies = []
    for j in range(PAIRS_PER_W):
        p_local = sid * PAIRS_PER_W + j
        f = p_local // D_PER_CORE
        dl = p_local % D_PER_CORE
        prow = f * D + cid * D_PER_CORE + dl
        copies.append(pltpu.async_copy(tbl_hbm.at[prow], trows_v.at[j], sem))
        copies.append(pltpu.async_copy(xt_hbm.at[f], xrows_v.at[j], sem))

    def z_body(k, _):
        o = k * 16
        for r in range(D_PER_CORE):
            vs_v[r, pl.ds(o, 16)] = zero16
            vq_v[r, pl.ds(o, 16)] = zero16
        return 0

    lax.fori_loop(0, CHUNKS, z_body, 0, unroll=4)
    for c in copies:
        c.wait()

    def pair_body(j, _):
        p_local = sid * PAIRS_PER_W + j
        dl = p_local % D_PER_CORE

        def chunk_body(k, _):
            o = k * 16
            idx = xrows_v[j, pl.ds(o, 16)]
            e = plsc.load_gather(trows_v.at[j], [idx])
            embuf_v[j, pl.ds(o, 16)] = e
            vs_v[dl, pl.ds(o, 16)] = vs_v[dl, pl.ds(o, 16)] + e
            vq_v[dl, pl.ds(o, 16)] = vq_v[dl, pl.ds(o, 16)] + e * e
            return 0

        lax.fori_loop(0, CHUNKS, chunk_body, 0, unroll=4)
        return 0

    lax.fori_loop(0, PAIRS_PER_W, pair_body, 0)

    wcopies = []
    for j in range(PAIRS_PER_W):
        p_local = sid * PAIRS_PER_W + j
        f = p_local // D_PER_CORE
        dl = p_local % D_PER_CORE
        prow = f * D + cid * D_PER_CORE + dl
        wcopies.append(pltpu.async_copy(embuf_v.at[j], emb_hbm.at[prow], sem))

    pltpu.sync_copy(vs_v, shs_v.at[sid])
    pltpu.sync_copy(vq_v, shq_v.at[sid])
    plsc.subcore_barrier()

    def za_body(k, _):
        o = k * 16
        for r in range(D_PER_CORE):
            accs_v[r, pl.ds(o, 16)] = zero16
            accq_v[r, pl.ds(o, 16)] = zero16
        return 0

    lax.fori_loop(0, COLS_PER_W // 16, za_body, 0, unroll=4)

    col0 = sid * COLS_PER_W

    def red_body(t, _):
        pltpu.sync_copy(shs_v.at[t, :, pl.ds(col0, COLS_PER_W)], tmp_v)
        for r in range(D_PER_CORE):
            for k2 in range(COLS_PER_W // 16):
                o = k2 * 16
                accs_v[r, pl.ds(o, 16)] = (
                    accs_v[r, pl.ds(o, 16)] + tmp_v[r, pl.ds(o, 16)])
        pltpu.sync_copy(shq_v.at[t, :, pl.ds(col0, COLS_PER_W)], tmp_v)
        for r in range(D_PER_CORE):
            for k2 in range(COLS_PER_W // 16):
                o = k2 * 16
                accq_v[r, pl.ds(o, 16)] = (
                    accq_v[r, pl.ds(o, 16)] + tmp_v[r, pl.ds(o, 16)])
        return 0

    lax.fori_loop(0, NSUB, red_body, 0)

    for r in range(D_PER_CORE):
        for k2 in range(COLS_PER_W // 16):
            o = k2 * 16
            sv = accs_v[r, pl.ds(o, 16)]
            qv = accq_v[r, pl.ds(o, 16)]
            tmp_v[r, pl.ds(o, 16)] = sv * sv - qv
    for r in range(D_PER_CORE):
        pltpu.sync_copy(tmp_v.at[r], fmp_hbm.at[cid * D_PER_CORE + r,
                                                pl.ds(col0, COLS_PER_W)])
    for c in wcopies:
        c.wait()


def _make_sc_call(interpret=False):
    return pl.kernel(
        _sc_body,
        out_type=[
            jax.ShapeDtypeStruct((NF * D, B), jnp.float32),
            jax.ShapeDtypeStruct((D, B), jnp.float32),
        ],
        mesh=_sc_mesh,
        compiler_params=pltpu.CompilerParams(
            use_tc_tiling_on_sc=False, needs_layout_passes=False),
        scratch_types=[
            pltpu.VMEM((PAIRS_PER_W, V), jnp.float32),
            pltpu.VMEM((PAIRS_PER_W, B), jnp.int32),
            pltpu.VMEM((PAIRS_PER_W, B), jnp.float32),
            pltpu.VMEM((D_PER_CORE, B), jnp.float32),
            pltpu.VMEM((D_PER_CORE, B), jnp.float32),
            pltpu.VMEM((D_PER_CORE, COLS_PER_W), jnp.float32),
            pltpu.VMEM((D_PER_CORE, COLS_PER_W), jnp.float32),
            pltpu.VMEM((D_PER_CORE, COLS_PER_W), jnp.float32),
            pltpu.VMEM_SHARED((NSUB, D_PER_CORE, B), jnp.float32),
            pltpu.VMEM_SHARED((NSUB, D_PER_CORE, B), jnp.float32),
            pltpu.SemaphoreType.DMA,
        ],
        interpret=interpret,
    )


_sc_gather_fm = _make_sc_call()


S_BLK = 1000
N_S_BLKS = S // (2 * S_BLK)


def _lin_body(xa_ref, xb_ref, wa_ref, wb_ref, out_ref):
    i = pl.program_id(0)
    pa = jnp.sum(xa_ref[...].astype(jnp.float32)
                 * lax.transpose(wa_ref[0], (1, 0)), axis=0)
    pb = jnp.sum(xb_ref[...].astype(jnp.float32)
                 * lax.transpose(wb_ref[0], (1, 0)), axis=0)
    p = (pa + pb)[None, :]

    @pl.when(i == 0)
    def _():
        out_ref[...] = p

    @pl.when(i != 0)
    def _():
        out_ref[...] += p


_lin_call = pl.pallas_call(
    _lin_body,
    grid=(N_S_BLKS,),
    in_specs=[
        pl.BlockSpec((S_BLK, B), lambda i: (i, 0)),
        pl.BlockSpec((S_BLK, B), lambda i: (i + N_S_BLKS, 0)),
        pl.BlockSpec((1, 1, S_BLK), lambda i: (i, 0, 0)),
        pl.BlockSpec((1, 1, S_BLK), lambda i: (i + N_S_BLKS, 0, 0)),
    ],
    out_specs=pl.BlockSpec((1, B), lambda i: (0, 0)),
    out_shape=jax.ShapeDtypeStruct((1, B), jnp.float32),
)


def _comb_body(lin_ref, fmp_ref, b_ref, out_ref):
    fm = 0.5 * jnp.sum(fmp_ref[...], axis=0, keepdims=True)
    tot = lin_ref[...] + fm + b_ref[0, 0]
    out_ref[...] = lax.transpose(tot, (1, 0))


_comb_call = pl.pallas_call(
    _comb_body,
    in_specs=[
        pl.BlockSpec(memory_space=pltpu.VMEM),
        pl.BlockSpec(memory_space=pltpu.VMEM),
        pl.BlockSpec(memory_space=pltpu.SMEM),
    ],
    out_specs=pl.BlockSpec(memory_space=pltpu.VMEM),
    out_shape=jax.ShapeDtypeStruct((B, 1), jnp.float32),
)


def kernel(x, tables, linear_W, linear_b):
    xt26 = x[:, :NF].T
    tbl = tables.transpose(0, 2, 1).reshape(NF * D, V)

    emb_t, fmp_t = _sc_gather_fm(xt26, tbl)

    xt = x.T
    wv = linear_W.T.reshape(2 * N_S_BLKS, 1, S_BLK)
    lin_t = _lin_call(xt, xt, wv, wv)

    out1 = _comb_call(lin_t, fmp_t, linear_b.reshape(1, 1))
    embeds = emb_t.reshape(NF, D, B).transpose(2, 0, 1)
    return (out1, embeds)


# speedup vs baseline: 1.0379x; 1.0379x over previous
import functools

import jax
import jax.numpy as jnp
from jax import lax
from jax.experimental import pallas as pl
from jax.experimental.pallas import tpu as pltpu
from jax.experimental.pallas import tpu_sc as plsc

B = 1024
NF = 26
V = 1000
D = 16
S = NF * V

NCORE = 2
NSUB = 16
D_PER_CORE = D // NCORE
PAIRS_PER_CORE = NF * D_PER_CORE
PAIRS_PER_W = PAIRS_PER_CORE // NSUB
CHUNKS = B // 16
COLS_PER_W = B // NSUB

_sc_mesh = plsc.VectorSubcoreMesh(core_axis_name="c", subcore_axis_name="s")


def _sc_body(xt_hbm, tbl_hbm, emb_hbm, fmp_hbm,
             trows_v, xrows_v, embuf_v, vs_v, vq_v,
             accs_v, accq_v, tmp_v, shs_v, shq_v, sem):
    cid = lax.axis_index("c")
    sid = lax.axis_index("s")
    zero16 = jnp.zeros((16,), jnp.float32)

    copies = []
    for j in range(PAIRS_PER_W):
        p_local = sid * PAIRS_PER_W + j
        f = p_local // D_PER_CORE
        dl = p_local % D_PER_CORE
        prow = f * D + cid * D_PER_CORE + dl
        copies.append(pltpu.async_copy(tbl_hbm.at[prow], trows_v.at[j], sem))
        copies.append(pltpu.async_copy(xt_hbm.at[f], xrows_v.at[j], sem))

    def z_body(k, _):
        o = k * 16
        for r in range(D_PER_CORE):
            vs_v[r, pl.ds(o, 16)] = zero16
            vq_v[r, pl.ds(o, 16)] = zero16
        return 0

    lax.fori_loop(0, CHUNKS, z_body, 0, unroll=4)
    for c in copies:
        c.wait()

    def pair_body(j, _):
        p_local = sid * PAIRS_PER_W + j
        dl = p_local % D_PER_CORE

        def chunk_body(k, _):
            o = k * 16
            idx = xrows_v[j, pl.ds(o, 16)]
            e = plsc.load_gather(trows_v.at[j], [idx])
            embuf_v[j, pl.ds(o, 16)] = e
            vs_v[dl, pl.ds(o, 16)] = vs_v[dl, pl.ds(o, 16)] + e
            vq_v[dl, pl.ds(o, 16)] = vq_v[dl, pl.ds(o, 16)] + e * e
            return 0

        lax.fori_loop(0, CHUNKS, chunk_body, 0, unroll=4)
        return 0

    lax.fori_loop(0, PAIRS_PER_W, pair_body, 0)

    wcopies = []
    for j in range(PAIRS_PER_W):
        p_local = sid * PAIRS_PER_W + j
        f = p_local // D_PER_CORE
        dl = p_local % D_PER_CORE
        prow = f * D + cid * D_PER_CORE + dl
        wcopies.append(pltpu.async_copy(embuf_v.at[j], emb_hbm.at[prow], sem))

    pltpu.sync_copy(vs_v, shs_v.at[sid])
    pltpu.sync_copy(vq_v, shq_v.at[sid])
    plsc.subcore_barrier()

    def za_body(k, _):
        o = k * 16
        for r in range(D_PER_CORE):
            accs_v[r, pl.ds(o, 16)] = zero16
            accq_v[r, pl.ds(o, 16)] = zero16
        return 0

    lax.fori_loop(0, COLS_PER_W // 16, za_body, 0, unroll=4)

    col0 = sid * COLS_PER_W

    def red_body(t, _):
        pltpu.sync_copy(shs_v.at[t, :, pl.ds(col0, COLS_PER_W)], tmp_v)
        for r in range(D_PER_CORE):
            for k2 in range(COLS_PER_W // 16):
                o = k2 * 16
                accs_v[r, pl.ds(o, 16)] = (
                    accs_v[r, pl.ds(o, 16)] + tmp_v[r, pl.ds(o, 16)])
        pltpu.sync_copy(shq_v.at[t, :, pl.ds(col0, COLS_PER_W)], tmp_v)
        for r in range(D_PER_CORE):
            for k2 in range(COLS_PER_W // 16):
                o = k2 * 16
                accq_v[r, pl.ds(o, 16)] = (
                    accq_v[r, pl.ds(o, 16)] + tmp_v[r, pl.ds(o, 16)])
        return 0

    lax.fori_loop(0, NSUB, red_body, 0)

    for r in range(D_PER_CORE):
        for k2 in range(COLS_PER_W // 16):
            o = k2 * 16
            sv = accs_v[r, pl.ds(o, 16)]
            qv = accq_v[r, pl.ds(o, 16)]
            tmp_v[r, pl.ds(o, 16)] = sv * sv - qv
    for r in range(D_PER_CORE):
        pltpu.sync_copy(tmp_v.at[r], fmp_hbm.at[cid * D_PER_CORE + r,
                                                pl.ds(col0, COLS_PER_W)])
    for c in wcopies:
        c.wait()


def _make_sc_call(interpret=False):
    return pl.kernel(
        _sc_body,
        out_type=[
            jax.ShapeDtypeStruct((NF * D, B), jnp.float32),
            jax.ShapeDtypeStruct((D, B), jnp.float32),
        ],
        mesh=_sc_mesh,
        compiler_params=pltpu.CompilerParams(
            use_tc_tiling_on_sc=False, needs_layout_passes=False),
        scratch_types=[
            pltpu.VMEM((PAIRS_PER_W, V), jnp.float32),
            pltpu.VMEM((PAIRS_PER_W, B), jnp.int32),
            pltpu.VMEM((PAIRS_PER_W, B), jnp.float32),
            pltpu.VMEM((D_PER_CORE, B), jnp.float32),
            pltpu.VMEM((D_PER_CORE, B), jnp.float32),
            pltpu.VMEM((D_PER_CORE, COLS_PER_W), jnp.float32),
            pltpu.VMEM((D_PER_CORE, COLS_PER_W), jnp.float32),
            pltpu.VMEM((D_PER_CORE, COLS_PER_W), jnp.float32),
            pltpu.VMEM_SHARED((NSUB, D_PER_CORE, B), jnp.float32),
            pltpu.VMEM_SHARED((NSUB, D_PER_CORE, B), jnp.float32),
            pltpu.SemaphoreType.DMA,
        ],
        interpret=interpret,
    )


_sc_gather_fm = _make_sc_call()


S_BLK = 2000
N_S_BLKS = S // S_BLK


def _lin_body(x_ref, w_ref, out_ref):
    i = pl.program_id(0)
    xf = x_ref[...].astype(jnp.float32)
    wcol = lax.transpose(w_ref[0], (1, 0))
    p = jnp.sum(xf * wcol, axis=0)[None, :]

    @pl.when(i == 0)
    def _():
        out_ref[...] = p

    @pl.when(i != 0)
    def _():
        out_ref[...] += p


_lin_call = pl.pallas_call(
    _lin_body,
    grid=(N_S_BLKS,),
    in_specs=[
        pl.BlockSpec((S_BLK, B), lambda i: (i, 0)),
        pl.BlockSpec((1, 1, S_BLK), lambda i: (i, 0, 0)),
    ],
    out_specs=pl.BlockSpec((1, B), lambda i: (0, 0)),
    out_shape=jax.ShapeDtypeStruct((1, B), jnp.float32),
)


def _comb_body(lin_ref, fmp_ref, b_ref, out_ref):
    fm = 0.5 * jnp.sum(fmp_ref[...], axis=0, keepdims=True)
    out_ref[...] = lin_ref[...] + fm + b_ref[0, 0]


_comb_call = pl.pallas_call(
    _comb_body,
    in_specs=[
        pl.BlockSpec(memory_space=pltpu.VMEM),
        pl.BlockSpec(memory_space=pltpu.VMEM),
        pl.BlockSpec(memory_space=pltpu.SMEM),
    ],
    out_specs=pl.BlockSpec(memory_space=pltpu.VMEM),
    out_shape=jax.ShapeDtypeStruct((1, B), jnp.float32),
)


def kernel(x, tables, linear_W, linear_b):
    xt26 = x[:, :NF].T
    tbl = tables.transpose(0, 2, 1).reshape(NF * D, V)

    emb_t, fmp_t = _sc_gather_fm(xt26, tbl)

    lin_t = _lin_call(x.T, linear_W.T.reshape(N_S_BLKS, 1, S_BLK))

    out1 = _comb_call(lin_t, fmp_t, linear_b.reshape(1, 1)).reshape(B, 1)
    embeds = emb_t.reshape(NF, D, B).transpose(2, 0, 1)
    return (out1, embeds)


# speedup vs baseline: 1.0384x; 1.0006x over previous
import functools

import jax
import jax.numpy as jnp
from jax import lax
from jax.experimental import pallas as pl
from jax.experimental.pallas import tpu as pltpu
from jax.experimental.pallas import tpu_sc as plsc

B = 1024
NF = 26
V = 1000
D = 16
S = NF * V

NCORE = 2
NSUB = 16
D_PER_CORE = D // NCORE
PAIRS_PER_CORE = NF * D_PER_CORE
PAIRS_PER_W = PAIRS_PER_CORE // NSUB
CHUNKS = B // 16
COLS_PER_W = B // NSUB

_sc_mesh = plsc.VectorSubcoreMesh(core_axis_name="c", subcore_axis_name="s")


def _sc_body(xt_hbm, tbl_hbm, emb_hbm, fmp_hbm,
             trows_v, xrows_v, embuf_v, vs_v, vq_v,
             accs_v, accq_v, tmp_v, shs_v, shq_v, sem):
    cid = lax.axis_index("c")
    sid = lax.axis_index("s")
    zero16 = jnp.zeros((16,), jnp.float32)

    copies = []
    for j in range(PAIRS_PER_W):
        p_local = sid * PAIRS_PER_W + j
        f = p_local // D_PER_CORE
        dl = p_local % D_PER_CORE
        prow = f * D + cid * D_PER_CORE + dl
        copies.append(pltpu.async_copy(tbl_hbm.at[prow], trows_v.at[j], sem))
        copies.append(pltpu.async_copy(xt_hbm.at[f], xrows_v.at[j], sem))

    def z_body(k, _):
        o = k * 16
        for r in range(D_PER_CORE):
            vs_v[r, pl.ds(o, 16)] = zero16
            vq_v[r, pl.ds(o, 16)] = zero16
        return 0

    lax.fori_loop(0, CHUNKS, z_body, 0, unroll=4)
    for c in copies:
        c.wait()

    def pair_body(j, _):
        p_local = sid * PAIRS_PER_W + j
        dl = p_local % D_PER_CORE

        def chunk_body(k, _):
            o = k * 16
            idx = xrows_v[j, pl.ds(o, 16)]
            e = plsc.load_gather(trows_v.at[j], [idx])
            embuf_v[j, pl.ds(o, 16)] = e
            vs_v[dl, pl.ds(o, 16)] = vs_v[dl, pl.ds(o, 16)] + e
            vq_v[dl, pl.ds(o, 16)] = vq_v[dl, pl.ds(o, 16)] + e * e
            return 0

        lax.fori_loop(0, CHUNKS, chunk_body, 0, unroll=4)
        return 0

    lax.fori_loop(0, PAIRS_PER_W, pair_body, 0)

    wcopies = []
    for j in range(PAIRS_PER_W):
        p_local = sid * PAIRS_PER_W + j
        f = p_local // D_PER_CORE
        dl = p_local % D_PER_CORE
        prow = f * D + cid * D_PER_CORE + dl
        wcopies.append(pltpu.async_copy(embuf_v.at[j], emb_hbm.at[prow], sem))

    pltpu.sync_copy(vs_v, shs_v.at[sid])
    pltpu.sync_copy(vq_v, shq_v.at[sid])
    plsc.subcore_barrier()

    def za_body(k, _):
        o = k * 16
        for r in range(D_PER_CORE):
            accs_v[r, pl.ds(o, 16)] = zero16
            accq_v[r, pl.ds(o, 16)] = zero16
        return 0

    lax.fori_loop(0, COLS_PER_W // 16, za_body, 0, unroll=4)

    col0 = sid * COLS_PER_W

    def red_body(t, _):
        pltpu.sync_copy(shs_v.at[t, :, pl.ds(col0, COLS_PER_W)], tmp_v)
        for r in range(D_PER_CORE):
            for k2 in range(COLS_PER_W // 16):
                o = k2 * 16
                accs_v[r, pl.ds(o, 16)] = (
                    accs_v[r, pl.ds(o, 16)] + tmp_v[r, pl.ds(o, 16)])
        pltpu.sync_copy(shq_v.at[t, :, pl.ds(col0, COLS_PER_W)], tmp_v)
        for r in range(D_PER_CORE):
            for k2 in range(COLS_PER_W // 16):
                o = k2 * 16
                accq_v[r, pl.ds(o, 16)] = (
                    accq_v[r, pl.ds(o, 16)] + tmp_v[r, pl.ds(o, 16)])
        return 0

    lax.fori_loop(0, NSUB, red_body, 0)

    for r in range(D_PER_CORE):
        for k2 in range(COLS_PER_W // 16):
            o = k2 * 16
            sv = accs_v[r, pl.ds(o, 16)]
            qv = accq_v[r, pl.ds(o, 16)]
            tmp_v[r, pl.ds(o, 16)] = sv * sv - qv
    for r in range(D_PER_CORE):
        pltpu.sync_copy(tmp_v.at[r], fmp_hbm.at[cid * D_PER_CORE + r,
                                                pl.ds(col0, COLS_PER_W)])
    for c in wcopies:
        c.wait()


def _make_sc_call():
    return pl.kernel(
        _sc_body,
        out_type=[
            jax.ShapeDtypeStruct((NF * D, B), jnp.float32),
            jax.ShapeDtypeStruct((D, B), jnp.float32),
        ],
        mesh=_sc_mesh,
        compiler_params=pltpu.CompilerParams(
            use_tc_tiling_on_sc=False, needs_layout_passes=False),
        scratch_types=[
            pltpu.VMEM((PAIRS_PER_W, V), jnp.float32),
            pltpu.VMEM((PAIRS_PER_W, B), jnp.int32),
            pltpu.VMEM((PAIRS_PER_W, B), jnp.float32),
            pltpu.VMEM((D_PER_CORE, B), jnp.float32),
            pltpu.VMEM((D_PER_CORE, B), jnp.float32),
            pltpu.VMEM((D_PER_CORE, COLS_PER_W), jnp.float32),
            pltpu.VMEM((D_PER_CORE, COLS_PER_W), jnp.float32),
            pltpu.VMEM((D_PER_CORE, COLS_PER_W), jnp.float32),
            pltpu.VMEM_SHARED((NSUB, D_PER_CORE, B), jnp.float32),
            pltpu.VMEM_SHARED((NSUB, D_PER_CORE, B), jnp.float32),
            pltpu.SemaphoreType.DMA,
        ],
    )


_sc_gather_fm = _make_sc_call()


S_BLK = 2000
N_S_BLKS = S // S_BLK


def _lin_body(x_ref, w_ref, out_ref):
    i = pl.program_id(0)
    xf = x_ref[...].astype(jnp.float32)
    wcol = lax.transpose(w_ref[0], (1, 0))
    p = jnp.sum(xf * wcol, axis=0)[None, :]

    @pl.when(i == 0)
    def _():
        out_ref[...] = p

    @pl.when(i != 0)
    def _():
        out_ref[...] += p


_lin_call = pl.pallas_call(
    _lin_body,
    grid=(N_S_BLKS,),
    in_specs=[
        pl.BlockSpec((S_BLK, B), lambda i: (i, 0)),
        pl.BlockSpec((1, 1, S_BLK), lambda i: (i, 0, 0)),
    ],
    out_specs=pl.BlockSpec((1, B), lambda i: (0, 0)),
    out_shape=jax.ShapeDtypeStruct((1, B), jnp.float32),
)


def _comb_body(lin_ref, fmp_ref, b_ref, out_ref):
    fm = 0.5 * jnp.sum(fmp_ref[...], axis=0, keepdims=True)
    out_ref[...] = lin_ref[...] + fm + b_ref[0, 0]


_comb_call = pl.pallas_call(
    _comb_body,
    in_specs=[
        pl.BlockSpec(memory_space=pltpu.VMEM),
        pl.BlockSpec(memory_space=pltpu.VMEM),
        pl.BlockSpec(memory_space=pltpu.SMEM),
    ],
    out_specs=pl.BlockSpec(memory_space=pltpu.VMEM),
    out_shape=jax.ShapeDtypeStruct((1, B), jnp.float32),
)


def kernel(x, tables, linear_W, linear_b):
    xt26 = x[:, :NF].T
    tbl = tables.transpose(0, 2, 1).reshape(NF * D, V)

    emb_t, fmp_t = _sc_gather_fm(xt26, tbl)

    lin_t = _lin_call(x.T, linear_W.T.reshape(N_S_BLKS, 1, S_BLK))

    out1 = _comb_call(lin_t, fmp_t, linear_b.reshape(1, 1)).reshape(B, 1)
    embeds = emb_t.reshape(NF, D, B).transpose(2, 0, 1)
    return (out1, embeds)


# speedup vs baseline: 1.0403x; 1.0018x over previous
import jax
import jax.numpy as jnp
from jax import lax
from jax.experimental import pallas as pl
from jax.experimental.pallas import tpu as pltpu
from jax.experimental.pallas import tpu_sc as plsc

B = 1024
NF = 26
V = 1000
D = 16
S = NF * V

NCORE = 2
NSUB = 16
D_PER_CORE = D // NCORE
PAIRS_PER_CORE = NF * D_PER_CORE
PAIRS_PER_W = PAIRS_PER_CORE // NSUB
CHUNKS = B // 16
COLS_PER_W = B // NSUB

_sc_mesh = plsc.VectorSubcoreMesh(core_axis_name="c", subcore_axis_name="s")


def _sc_body(xt_hbm, tbl_hbm, emb_hbm, fmp_hbm,
             trows_v, xrows_v, embuf_v, vs_v, vq_v,
             accs_v, accq_v, tmp_v, shs_v, shq_v, sem):
    cid = lax.axis_index("c")
    sid = lax.axis_index("s")
    zero16 = jnp.zeros((16,), jnp.float32)

    copies = []
    for j in range(PAIRS_PER_W):
        p_local = sid * PAIRS_PER_W + j
        f = p_local // D_PER_CORE
        dl = p_local % D_PER_CORE
        prow = f * D + cid * D_PER_CORE + dl
        copies.append(pltpu.async_copy(tbl_hbm.at[prow], trows_v.at[j], sem))
        copies.append(pltpu.async_copy(xt_hbm.at[f], xrows_v.at[j], sem))

    def z_body(k, _):
        o = k * 16
        for r in range(D_PER_CORE):
            vs_v[r, pl.ds(o, 16)] = zero16
            vq_v[r, pl.ds(o, 16)] = zero16
        return 0

    lax.fori_loop(0, CHUNKS, z_body, 0, unroll=4)
    for c in copies:
        c.wait()

    def pair_body(j, _):
        p_local = sid * PAIRS_PER_W + j
        dl = p_local % D_PER_CORE

        def chunk_body(k, _):
            o = k * 16
            idx = xrows_v[j, pl.ds(o, 16)]
            e = plsc.load_gather(trows_v.at[j], [idx])
            embuf_v[j, pl.ds(o, 16)] = e
            vs_v[dl, pl.ds(o, 16)] = vs_v[dl, pl.ds(o, 16)] + e
            vq_v[dl, pl.ds(o, 16)] = vq_v[dl, pl.ds(o, 16)] + e * e
            return 0

        lax.fori_loop(0, CHUNKS, chunk_body, 0, unroll=4)
        return 0

    lax.fori_loop(0, PAIRS_PER_W, pair_body, 0)

    wcopies = []
    for j in range(PAIRS_PER_W):
        p_local = sid * PAIRS_PER_W + j
        f = p_local // D_PER_CORE
        dl = p_local % D_PER_CORE
        prow = f * D + cid * D_PER_CORE + dl
        wcopies.append(pltpu.async_copy(embuf_v.at[j], emb_hbm.at[prow], sem))

    pltpu.sync_copy(vs_v, shs_v.at[sid])
    pltpu.sync_copy(vq_v, shq_v.at[sid])
    plsc.subcore_barrier()

    def za_body(k, _):
        o = k * 16
        for r in range(D_PER_CORE):
            accs_v[r, pl.ds(o, 16)] = zero16
            accq_v[r, pl.ds(o, 16)] = zero16
        return 0

    lax.fori_loop(0, COLS_PER_W // 16, za_body, 0, unroll=4)

    col0 = sid * COLS_PER_W

    def red_body(t, _):
        pltpu.sync_copy(shs_v.at[t, :, pl.ds(col0, COLS_PER_W)], tmp_v)
        for r in range(D_PER_CORE):
            for k2 in range(COLS_PER_W // 16):
                o = k2 * 16
                accs_v[r, pl.ds(o, 16)] = (
                    accs_v[r, pl.ds(o, 16)] + tmp_v[r, pl.ds(o, 16)])
        pltpu.sync_copy(shq_v.at[t, :, pl.ds(col0, COLS_PER_W)], tmp_v)
        for r in range(D_PER_CORE):
            for k2 in range(COLS_PER_W // 16):
                o = k2 * 16
                accq_v[r, pl.ds(o, 16)] = (
                    accq_v[r, pl.ds(o, 16)] + tmp_v[r, pl.ds(o, 16)])
        return 0

    lax.fori_loop(0, NSUB, red_body, 0)

    for r in range(D_PER_CORE):
        for k2 in range(COLS_PER_W // 16):
            o = k2 * 16
            sv = accs_v[r, pl.ds(o, 16)]
            qv = accq_v[r, pl.ds(o, 16)]
            tmp_v[r, pl.ds(o, 16)] = sv * sv - qv
    for r in range(D_PER_CORE):
        pltpu.sync_copy(tmp_v.at[r], fmp_hbm.at[cid * D_PER_CORE + r,
                                                pl.ds(col0, COLS_PER_W)])
    for c in wcopies:
        c.wait()


def _make_sc_call():
    return pl.kernel(
        _sc_body,
        out_type=[
            jax.ShapeDtypeStruct((NF * D, B), jnp.float32),
            jax.ShapeDtypeStruct((D, B), jnp.float32),
        ],
        mesh=_sc_mesh,
        compiler_params=pltpu.CompilerParams(
            use_tc_tiling_on_sc=False, needs_layout_passes=False),
        scratch_types=[
            pltpu.VMEM((PAIRS_PER_W, V), jnp.float32),
            pltpu.VMEM((PAIRS_PER_W, B), jnp.int32),
            pltpu.VMEM((PAIRS_PER_W, B), jnp.float32),
            pltpu.VMEM((D_PER_CORE, B), jnp.float32),
            pltpu.VMEM((D_PER_CORE, B), jnp.float32),
            pltpu.VMEM((D_PER_CORE, COLS_PER_W), jnp.float32),
            pltpu.VMEM((D_PER_CORE, COLS_PER_W), jnp.float32),
            pltpu.VMEM((D_PER_CORE, COLS_PER_W), jnp.float32),
            pltpu.VMEM_SHARED((NSUB, D_PER_CORE, B), jnp.float32),
            pltpu.VMEM_SHARED((NSUB, D_PER_CORE, B), jnp.float32),
            pltpu.SemaphoreType.DMA,
        ],
    )


_sc_gather_fm = _make_sc_call()


S_BLK = 2000
N_S_BLKS = S // S_BLK


def _lin_body(x_ref, w_ref, out_ref):
    i = pl.program_id(0)
    xf = x_ref[...].astype(jnp.float32)
    wcol = lax.transpose(w_ref[0], (1, 0))
    p = jnp.sum(xf * wcol, axis=0)[None, :]

    @pl.when(i == 0)
    def _():
        out_ref[...] = p

    @pl.when(i != 0)
    def _():
        out_ref[...] += p


_lin_call = pl.pallas_call(
    _lin_body,
    grid=(N_S_BLKS,),
    in_specs=[
        pl.BlockSpec((S_BLK, B), lambda i: (i, 0)),
        pl.BlockSpec((1, 1, S_BLK), lambda i: (i, 0, 0)),
    ],
    out_specs=pl.BlockSpec((1, B), lambda i: (0, 0)),
    out_shape=jax.ShapeDtypeStruct((1, B), jnp.float32),
)


def _comb_body(lin_ref, fmp_ref, b_ref, out_ref):
    fm = 0.5 * jnp.sum(fmp_ref[...], axis=0, keepdims=True)
    out_ref[...] = lin_ref[...] + fm + b_ref[0, 0]


_comb_call = pl.pallas_call(
    _comb_body,
    in_specs=[
        pl.BlockSpec(memory_space=pltpu.VMEM),
        pl.BlockSpec(memory_space=pltpu.VMEM),
        pl.BlockSpec(memory_space=pltpu.SMEM),
    ],
    out_specs=pl.BlockSpec(memory_space=pltpu.VMEM),
    out_shape=jax.ShapeDtypeStruct((1, B), jnp.float32),
)


def kernel(x, tables, linear_W, linear_b):
    xt26 = x[:, :NF].T
    tbl = tables.transpose(0, 2, 1).reshape(NF * D, V)

    emb_t, fmp_t = _sc_gather_fm(xt26, tbl)

    lin_t = _lin_call(x.T, linear_W.T.reshape(N_S_BLKS, 1, S_BLK))

    out1 = _comb_call(lin_t, fmp_t, linear_b.reshape(1, 1)).reshape(B, 1)
    embeds = emb_t.reshape(NF, D, B).transpose(2, 0, 1)
    return (out1, embeds)
